# baseline (speedup 1.0000x reference)
import jax
import jax.numpy as jnp
from jax import lax
from jax.experimental import pallas as pl
from jax.experimental.pallas import tpu as pltpu
from jax.experimental.pallas import tpu_sc as plsc

B, L, V, D = 4096, 20, 100000, 4
LANES = 16
NC, NS = 2, 16
ROWS_PER_W = B // NS
RCHUNKS = ROWS_PER_W // LANES


def _project_body(t1_ref, t2_ref, w_ref, o1_ref, o2_ref):
    w = w_ref[...]
    o1_ref[...] = jnp.sum(t1_ref[...] * w, axis=0)
    o2_ref[...] = jnp.sum(t2_ref[...] * w, axis=0)


def _pool_body(s1, s2, idx1, idx2, bvec, out, table_v, idx_v, out_v, b_v):
    c = lax.axis_index("c")
    s = lax.axis_index("s")
    base = s * ROWS_PER_W

    @pl.when(c == 0)
    def _():
        pltpu.sync_copy(s1, table_v)
        for j in range(L):
            pltpu.sync_copy(idx1.at[pl.ds(j * B + base, ROWS_PER_W)],
                            idx_v.at[pl.ds(j * ROWS_PER_W, ROWS_PER_W)])

    @pl.when(c == 1)
    def _():
        pltpu.sync_copy(s2, table_v)
        for j in range(L):
            pltpu.sync_copy(idx2.at[pl.ds(j * B + base, ROWS_PER_W)],
                            idx_v.at[pl.ds(j * ROWS_PER_W, ROWS_PER_W)])

    pltpu.sync_copy(bvec, b_v)
    bias = b_v[...]

    def rc_body(rc, carry):
        off = rc * LANES
        acc = bias
        for j in range(L):
            tidx = idx_v[pl.ds(j * ROWS_PER_W + off, LANES)]
            acc = acc + plsc.load_gather(table_v, [tidx])
        out_v[pl.ds(off, LANES)] = acc
        return carry

    lax.fori_loop(0, RCHUNKS, rc_body, 0)
    pltpu.sync_copy(out_v, out.at[pl.ds(c * B + base, ROWS_PER_W)])


def kernel(indices_f1, indices_f2, table_f1, table_f2, W, b):
    s1, s2 = pl.pallas_call(
        _project_body,
        out_shape=[jax.ShapeDtypeStruct((V,), jnp.float32)] * 2,
    )(table_f1.T, table_f2.T, W.T)
    bvec = jnp.tile(b, LANES)

    mesh = plsc.VectorSubcoreMesh(core_axis_name="c", subcore_axis_name="s")
    pool = pl.kernel(
        _pool_body,
        mesh=mesh,
        out_type=jax.ShapeDtypeStruct((2 * B,), jnp.float32),
        compiler_params=pltpu.CompilerParams(needs_layout_passes=False),
        scratch_types=[
            pltpu.VMEM((V,), jnp.float32),
            pltpu.VMEM((ROWS_PER_W * L,), jnp.int32),
            pltpu.VMEM((ROWS_PER_W,), jnp.float32),
            pltpu.VMEM((LANES,), jnp.float32),
        ],
    )
    out = pool(s1, s2, indices_f1.T.reshape(-1), indices_f2.T.reshape(-1),
               bvec)
    return out.reshape(2 * B, 1)


# speedup vs baseline: 1.2295x; 1.2295x over previous
import jax
import jax.numpy as jnp
from jax import lax
from jax.experimental import pallas as pl
from jax.experimental.pallas import tpu as pltpu
from jax.experimental.pallas import tpu_sc as plsc

B, L, V, D = 4096, 20, 100000, 4
LANES = 16
NC, NS = 2, 16
ROWS_PER_W = B // NS
RCHUNKS = ROWS_PER_W // LANES


def _project_body(t1_ref, t2_ref, w_ref, o1_ref, o2_ref):
    w = w_ref[...]
    o1_ref[...] = jnp.sum(t1_ref[...] * w, axis=0)
    o2_ref[...] = jnp.sum(t2_ref[...] * w, axis=0)


def _pool_body(s1, s2, idx1, idx2, bvec, out, table_v, idx_v, out_v, b_v,
               sem):
    c = lax.axis_index("c")
    s = lax.axis_index("s")
    base = s * ROWS_PER_W

    @pl.when(c == 0)
    def _():
        copies = [pltpu.async_copy(
            idx1.at[pl.ds(j * B + base, ROWS_PER_W)],
            idx_v.at[pl.ds(j * ROWS_PER_W, ROWS_PER_W)], sem)
            for j in range(L)]
        copies.append(pltpu.async_copy(s1, table_v, sem))
        for cp in copies:
            cp.wait()

    @pl.when(c == 1)
    def _():
        copies = [pltpu.async_copy(
            idx2.at[pl.ds(j * B + base, ROWS_PER_W)],
            idx_v.at[pl.ds(j * ROWS_PER_W, ROWS_PER_W)], sem)
            for j in range(L)]
        copies.append(pltpu.async_copy(s2, table_v, sem))
        for cp in copies:
            cp.wait()

    pltpu.sync_copy(bvec, b_v)
    bias = b_v[...]

    def rc_body(rc, carry):
        off = rc * LANES
        acc = bias
        for j in range(L):
            tidx = idx_v[pl.ds(j * ROWS_PER_W + off, LANES)]
            acc = acc + plsc.load_gather(table_v, [tidx])
        out_v[pl.ds(off, LANES)] = acc
        return carry

    lax.fori_loop(0, RCHUNKS, rc_body, 0)
    pltpu.sync_copy(out_v, out.at[pl.ds(c * B + base, ROWS_PER_W)])


def kernel(indices_f1, indices_f2, table_f1, table_f2, W, b):
    s1, s2 = pl.pallas_call(
        _project_body,
        out_shape=[jax.ShapeDtypeStruct((V,), jnp.float32)] * 2,
    )(table_f1.T, table_f2.T, W.T)
    bvec = jnp.tile(b, LANES)

    mesh = plsc.VectorSubcoreMesh(core_axis_name="c", subcore_axis_name="s")
    pool = pl.kernel(
        _pool_body,
        mesh=mesh,
        out_type=jax.ShapeDtypeStruct((2 * B,), jnp.float32),
        compiler_params=pltpu.CompilerParams(needs_layout_passes=False),
        scratch_types=[
            pltpu.VMEM((V,), jnp.float32),
            pltpu.VMEM((ROWS_PER_W * L,), jnp.int32),
            pltpu.VMEM((ROWS_PER_W,), jnp.float32),
            pltpu.VMEM((LANES,), jnp.float32),
            pltpu.SemaphoreType.DMA,
        ],
    )
    out = pool(s1, s2, indices_f1.T.reshape(-1), indices_f2.T.reshape(-1),
               bvec)
    return out.reshape(2 * B, 1)


# speedup vs baseline: 1.2997x; 1.0571x over previous
import jax
import jax.numpy as jnp
from jax import lax
from jax.experimental import pallas as pl
from jax.experimental.pallas import tpu as pltpu
from jax.experimental.pallas import tpu_sc as plsc

B, L, V, D = 4096, 20, 100000, 4
LANES = 16
NC, NS = 2, 16
ROWS_PER_W = B // NS
RCHUNKS = ROWS_PER_W // LANES


def _project_body(t1_ref, t2_ref, w_ref, o1_ref, o2_ref):
    w = w_ref[...]
    o1_ref[...] = jnp.sum(t1_ref[...] * w, axis=0)
    o2_ref[...] = jnp.sum(t2_ref[...] * w, axis=0)


def _pool_body(s1, s2, idx1, idx2, bvec, out, idx_v, vals_v, out_v, b_v,
               sem):
    c = lax.axis_index("c")
    s = lax.axis_index("s")
    base = s * ROWS_PER_W

    @pl.when(c == 0)
    def _():
        copies = [pltpu.async_copy(
            idx1.at[pl.ds(j * B + base, ROWS_PER_W)],
            idx_v.at[pl.ds(j * ROWS_PER_W, ROWS_PER_W)], sem)
            for j in range(L)]
        for cp in copies:
            cp.wait()
        pltpu.async_copy(s1.at[idx_v], vals_v, sem).wait()

    @pl.when(c == 1)
    def _():
        copies = [pltpu.async_copy(
            idx2.at[pl.ds(j * B + base, ROWS_PER_W)],
            idx_v.at[pl.ds(j * ROWS_PER_W, ROWS_PER_W)], sem)
            for j in range(L)]
        for cp in copies:
            cp.wait()
        pltpu.async_copy(s2.at[idx_v], vals_v, sem).wait()

    pltpu.sync_copy(bvec, b_v)
    bias = b_v[...]

    def rc_body(rc, carry):
        off = rc * LANES
        acc = bias
        for j in range(L):
            acc = acc + vals_v[pl.ds(j * ROWS_PER_W + off, LANES)]
        out_v[pl.ds(off, LANES)] = acc
        return carry

    lax.fori_loop(0, RCHUNKS, rc_body, 0)
    pltpu.sync_copy(out_v, out.at[pl.ds(c * B + base, ROWS_PER_W)])


def kernel(indices_f1, indices_f2, table_f1, table_f2, W, b):
    s1, s2 = pl.pallas_call(
        _project_body,
        out_shape=[jax.ShapeDtypeStruct((V,), jnp.float32)] * 2,
    )(table_f1.T, table_f2.T, W.T)
    bvec = jnp.tile(b, LANES)

    mesh = plsc.VectorSubcoreMesh(core_axis_name="c", subcore_axis_name="s")
    pool = pl.kernel(
        _pool_body,
        mesh=mesh,
        out_type=jax.ShapeDtypeStruct((2 * B,), jnp.float32),
        compiler_params=pltpu.CompilerParams(needs_layout_passes=False),
        scratch_types=[
            pltpu.VMEM((ROWS_PER_W * L,), jnp.int32),
            pltpu.VMEM((ROWS_PER_W * L,), jnp.float32),
            pltpu.VMEM((ROWS_PER_W,), jnp.float32),
            pltpu.VMEM((LANES,), jnp.float32),
            pltpu.SemaphoreType.DMA,
        ],
    )
    out = pool(s1, s2, indices_f1.T.reshape(-1), indices_f2.T.reshape(-1),
               bvec)
    return out.reshape(2 * B, 1)


# speedup vs baseline: 1.4137x; 1.0878x over previous
import jax
import jax.numpy as jnp
from jax import lax
from jax.experimental import pallas as pl
from jax.experimental.pallas import tpu as pltpu
from jax.experimental.pallas import tpu_sc as plsc

B, L, V, D = 4096, 20, 100000, 4
LANES = 16
NC, NS = 2, 16
ROWS_PER_W = B // NS
RCHUNKS = ROWS_PER_W // LANES


def _project_body(t1_ref, t2_ref, w_ref, b_ref, o1_ref, o2_ref, o3_ref):
    w = w_ref[...].reshape(1, D)
    o1_ref[...] = jnp.dot(w, t1_ref[...], preferred_element_type=jnp.float32)[0]
    o2_ref[...] = jnp.dot(w, t2_ref[...], preferred_element_type=jnp.float32)[0]
    o3_ref[...] = jnp.broadcast_to(b_ref[...], (LANES,))


def _pool_body(s1, s2, idx1, idx2, bvec, out, idx_v, vals_v, out_v, b_v,
               sem):
    c = lax.axis_index("c")
    s = lax.axis_index("s")
    base = s * ROWS_PER_W

    @pl.when(c == 0)
    def _():
        copies = [pltpu.async_copy(
            idx1.at[pl.ds(j * B + base, ROWS_PER_W)],
            idx_v.at[pl.ds(j * ROWS_PER_W, ROWS_PER_W)], sem)
            for j in range(L)]
        for cp in copies:
            cp.wait()
        pltpu.async_copy(s1.at[idx_v], vals_v, sem).wait()

    @pl.when(c == 1)
    def _():
        copies = [pltpu.async_copy(
            idx2.at[pl.ds(j * B + base, ROWS_PER_W)],
            idx_v.at[pl.ds(j * ROWS_PER_W, ROWS_PER_W)], sem)
            for j in range(L)]
        for cp in copies:
            cp.wait()
        pltpu.async_copy(s2.at[idx_v], vals_v, sem).wait()

    pltpu.sync_copy(bvec, b_v)
    bias = b_v[...]

    def rc_body(rc, carry):
        off = rc * LANES
        acc = bias
        for j in range(L):
            acc = acc + vals_v[pl.ds(j * ROWS_PER_W + off, LANES)]
        out_v[pl.ds(off, LANES)] = acc
        return carry

    lax.fori_loop(0, RCHUNKS, rc_body, 0)
    pltpu.sync_copy(out_v, out.at[pl.ds(c * B + base, ROWS_PER_W)])


def kernel(indices_f1, indices_f2, table_f1, table_f2, W, b):
    s1, s2, bvec = pl.pallas_call(
        _project_body,
        out_shape=[jax.ShapeDtypeStruct((V,), jnp.float32)] * 2
        + [jax.ShapeDtypeStruct((LANES,), jnp.float32)],
    )(table_f1.T, table_f2.T, W.reshape(D), b)

    mesh = plsc.VectorSubcoreMesh(core_axis_name="c", subcore_axis_name="s")
    pool = pl.kernel(
        _pool_body,
        mesh=mesh,
        out_type=jax.ShapeDtypeStruct((2 * B,), jnp.float32),
        compiler_params=pltpu.CompilerParams(needs_layout_passes=False),
        scratch_types=[
            pltpu.VMEM((ROWS_PER_W * L,), jnp.int32),
            pltpu.VMEM((ROWS_PER_W * L,), jnp.float32),
            pltpu.VMEM((ROWS_PER_W,), jnp.float32),
            pltpu.VMEM((LANES,), jnp.float32),
            pltpu.SemaphoreType.DMA,
        ],
    )
    out = pool(s1, s2, indices_f1.T.reshape(-1), indices_f2.T.reshape(-1),
               bvec)
    return out.reshape(2 * B, 1)


# speedup vs baseline: 1.5801x; 1.1177x over previous
import jax
import jax.numpy as jnp
from jax import lax
from jax.experimental import pallas as pl
from jax.experimental.pallas import tpu as pltpu
from jax.experimental.pallas import tpu_sc as plsc

B, L, V, D = 4096, 20, 100000, 4
LANES = 16
NC, NS = 2, 16
ROWS_PER_W = B // NS
RCHUNKS = ROWS_PER_W // LANES


def _project_body(t1_ref, t2_ref, w_ref, b_ref, i1_ref, i2_ref,
                  o1_ref, o2_ref, o3_ref, oi1_ref, oi2_ref):
    w = w_ref[...].reshape(1, D)
    o1_ref[...] = jnp.dot(w, t1_ref[...], preferred_element_type=jnp.float32)[0]
    o2_ref[...] = jnp.dot(w, t2_ref[...], preferred_element_type=jnp.float32)[0]
    o3_ref[...] = jnp.broadcast_to(b_ref[...], (LANES,))
    oi1_ref[...] = i1_ref[...].reshape(L * B)
    oi2_ref[...] = i2_ref[...].reshape(L * B)


def _pool_body(s1, s2, idx1, idx2, bvec, out, idx_v, vals_v, out_v, b_v,
               sem):
    c = lax.axis_index("c")
    s = lax.axis_index("s")
    base = s * ROWS_PER_W

    @pl.when(c == 0)
    def _():
        copies = [pltpu.async_copy(
            idx1.at[pl.ds(j * B + base, ROWS_PER_W)],
            idx_v.at[pl.ds(j * ROWS_PER_W, ROWS_PER_W)], sem)
            for j in range(L)]
        for cp in copies:
            cp.wait()
        pltpu.async_copy(s1.at[idx_v], vals_v, sem).wait()

    @pl.when(c == 1)
    def _():
        copies = [pltpu.async_copy(
            idx2.at[pl.ds(j * B + base, ROWS_PER_W)],
            idx_v.at[pl.ds(j * ROWS_PER_W, ROWS_PER_W)], sem)
            for j in range(L)]
        for cp in copies:
            cp.wait()
        pltpu.async_copy(s2.at[idx_v], vals_v, sem).wait()

    pltpu.sync_copy(bvec, b_v)
    bias = b_v[...]

    def rc_body(rc, carry):
        off = rc * LANES
        acc = bias
        for j in range(L):
            acc = acc + vals_v[pl.ds(j * ROWS_PER_W + off, LANES)]
        out_v[pl.ds(off, LANES)] = acc
        return carry

    lax.fori_loop(0, RCHUNKS, rc_body, 0)
    pltpu.sync_copy(out_v, out.at[pl.ds(c * B + base, ROWS_PER_W)])


def kernel(indices_f1, indices_f2, table_f1, table_f2, W, b):
    s1, s2, bvec, idx1f, idx2f = pl.pallas_call(
        _project_body,
        out_shape=[jax.ShapeDtypeStruct((V,), jnp.float32)] * 2
        + [jax.ShapeDtypeStruct((LANES,), jnp.float32)]
        + [jax.ShapeDtypeStruct((L * B,), jnp.int32)] * 2,
    )(table_f1.T, table_f2.T, W.reshape(D), b, indices_f1.T, indices_f2.T)

    mesh = plsc.VectorSubcoreMesh(core_axis_name="c", subcore_axis_name="s")
    pool = pl.kernel(
        _pool_body,
        mesh=mesh,
        out_type=jax.ShapeDtypeStruct((2 * B,), jnp.float32),
        compiler_params=pltpu.CompilerParams(needs_layout_passes=False),
        scratch_types=[
            pltpu.VMEM((ROWS_PER_W * L,), jnp.int32),
            pltpu.VMEM((ROWS_PER_W * L,), jnp.float32),
            pltpu.VMEM((ROWS_PER_W,), jnp.float32),
            pltpu.VMEM((LANES,), jnp.float32),
            pltpu.SemaphoreType.DMA,
        ],
    )
    out = pool(s1, s2, idx1f, idx2f, bvec)
    return out.reshape(2 * B, 1)


# speedup vs baseline: 1.7816x; 1.1275x over previous
import jax
import jax.numpy as jnp
from jax import lax
from jax.experimental import pallas as pl
from jax.experimental.pallas import tpu as pltpu
from jax.experimental.pallas import tpu_sc as plsc

B, L, V, D = 4096, 20, 100000, 4
LANES = 16
NC, NS = 2, 16
ROWS_PER_W = B // NS
RCHUNKS = ROWS_PER_W // LANES


def _project_body(t1_ref, t2_ref, w_ref, b_ref, i1_ref, i2_ref,
                  o1_ref, o2_ref, o3_ref, oi1_ref, oi2_ref):
    w = w_ref[...].reshape(1, D)
    o1_ref[...] = jnp.dot(w, t1_ref[...], preferred_element_type=jnp.float32)[0]
    o2_ref[...] = jnp.dot(w, t2_ref[...], preferred_element_type=jnp.float32)[0]
    o3_ref[...] = jnp.broadcast_to(b_ref[...], (LANES,))
    oi1_ref[...] = i1_ref[...].reshape(L * B)
    oi2_ref[...] = i2_ref[...].reshape(L * B)


def _pool_body(s1, s2, idx1, idx2, bvec, out, idx_v, vals_v, out_v, b_v,
               s_sh, sem):
    c = lax.axis_index("c")
    s = lax.axis_index("s")
    base = s * ROWS_PER_W

    @pl.when(jnp.logical_and(c == 0, s == 0))
    def _():
        pltpu.sync_copy(s1, s_sh)

    @pl.when(jnp.logical_and(c == 1, s == 0))
    def _():
        pltpu.sync_copy(s2, s_sh)

    @pl.when(c == 0)
    def _():
        copies = [pltpu.async_copy(
            idx1.at[pl.ds(j * B + base, ROWS_PER_W)],
            idx_v.at[pl.ds(j * ROWS_PER_W, ROWS_PER_W)], sem)
            for j in range(L)]
        for cp in copies:
            cp.wait()

    @pl.when(c == 1)
    def _():
        copies = [pltpu.async_copy(
            idx2.at[pl.ds(j * B + base, ROWS_PER_W)],
            idx_v.at[pl.ds(j * ROWS_PER_W, ROWS_PER_W)], sem)
            for j in range(L)]
        for cp in copies:
            cp.wait()

    plsc.subcore_barrier()
    pltpu.async_copy(s_sh.at[idx_v], vals_v, sem).wait()

    pltpu.sync_copy(bvec, b_v)
    bias = b_v[...]

    def rc_body(rc, carry):
        off = rc * LANES
        acc = bias
        for j in range(L):
            acc = acc + vals_v[pl.ds(j * ROWS_PER_W + off, LANES)]
        out_v[pl.ds(off, LANES)] = acc
        return carry

    lax.fori_loop(0, RCHUNKS, rc_body, 0)
    pltpu.sync_copy(out_v, out.at[pl.ds(c * B + base, ROWS_PER_W)])


def kernel(indices_f1, indices_f2, table_f1, table_f2, W, b):
    s1, s2, bvec, idx1f, idx2f = pl.pallas_call(
        _project_body,
        out_shape=[jax.ShapeDtypeStruct((V,), jnp.float32)] * 2
        + [jax.ShapeDtypeStruct((LANES,), jnp.float32)]
        + [jax.ShapeDtypeStruct((L * B,), jnp.int32)] * 2,
    )(table_f1.T, table_f2.T, W.reshape(D), b, indices_f1.T, indices_f2.T)

    mesh = plsc.VectorSubcoreMesh(core_axis_name="c", subcore_axis_name="s")
    pool = pl.kernel(
        _pool_body,
        mesh=mesh,
        out_type=jax.ShapeDtypeStruct((2 * B,), jnp.float32),
        compiler_params=pltpu.CompilerParams(needs_layout_passes=False),
        scratch_types=[
            pltpu.VMEM((ROWS_PER_W * L,), jnp.int32),
            pltpu.VMEM((ROWS_PER_W * L,), jnp.float32),
            pltpu.VMEM((ROWS_PER_W,), jnp.float32),
            pltpu.VMEM((LANES,), jnp.float32),
            pltpu.VMEM_SHARED((V,), jnp.float32),
            pltpu.SemaphoreType.DMA,
        ],
    )
    out = pool(s1, s2, idx1f, idx2f, bvec)
    return out.reshape(2 * B, 1)
